# baseline (speedup 1.0000x reference)
import jax
import jax.numpy as jnp
from jax.experimental import pallas as pl
from jax.experimental.pallas import tpu as pltpu

_ROWS = 65536
_COLS = 256
_STRIDE = 1024
_NB = _ROWS // _STRIDE


def _body(x_ref, o_ref):
    o_ref[...] = x_ref[...]
    o_ref[0:1, :] = jnp.maximum(x_ref[0:1, :], 0.0)


def kernel(x):
    return pl.pallas_call(
        _body,
        grid=(_NB,),
        in_specs=[pl.BlockSpec((8, _COLS), lambda i: (i * (_STRIDE // 8), 0))],
        out_specs=pl.BlockSpec((8, _COLS), lambda i: (i * (_STRIDE // 8), 0)),
        out_shape=jax.ShapeDtypeStruct((_ROWS, _COLS), jnp.float32),
        input_output_aliases={0: 0},
        compiler_params=pltpu.CompilerParams(
            dimension_semantics=("arbitrary",),
        ),
    )(x)


# speedup vs baseline: 1.7134x; 1.7134x over previous
import jax
import jax.numpy as jnp
from jax.experimental import pallas as pl
from jax.experimental.pallas import tpu as pltpu

_ROWS = 65536
_COLS = 256
_STRIDE = 1024
_BLOCK = 8192
_NBLOCKS = _ROWS // _BLOCK


def _body(x_ref, o_ref):
    o_ref[...] = x_ref[...]
    for r in range(0, _BLOCK, _STRIDE):
        o_ref[r:r + 1, :] = jnp.maximum(x_ref[r:r + 1, :], 0.0)


def kernel(x):
    return pl.pallas_call(
        _body,
        grid=(_NBLOCKS,),
        in_specs=[pl.BlockSpec((_BLOCK, _COLS), lambda i: (i, 0))],
        out_specs=pl.BlockSpec((_BLOCK, _COLS), lambda i: (i, 0)),
        out_shape=jax.ShapeDtypeStruct((_ROWS, _COLS), jnp.float32),
        compiler_params=pltpu.CompilerParams(
            dimension_semantics=("arbitrary",),
        ),
    )(x)
